# baseline (speedup 1.0000x reference)
import functools

import jax
import jax.numpy as jnp
from jax import lax
from jax.experimental import pallas as pl
from jax.experimental.pallas import tpu as pltpu
from jax.experimental.pallas import tpu_sc as plsc

O = 10000
T = 160000
D = 384
H = 384

NC = 2
NS = 16
NW = NC * NS

C = 128
GCH = 19
TA = NW * (GCH * C + 64)
TB = T - TA
PW = 128
NP = H // PW
CPT = 624
MT = 256


def _proj_table(obj, W1r):
    nO = 10
    bO = O // nO

    def body(w_ref, x_ref, o_ref):
        o_ref[...] = jnp.dot(x_ref[...], w_ref[0],
                             preferred_element_type=jnp.float32)

    return pl.pallas_call(
        body,
        grid=(2, nO),
        in_specs=[
            pl.BlockSpec((1, D, H), lambda g, j: (2 * g, 0, 0)),
            pl.BlockSpec((bO, D), lambda g, j: (j, 0)),
        ],
        out_specs=pl.BlockSpec((bO, H), lambda g, j: (g * nO + j, 0)),
        out_shape=jax.ShapeDtypeStruct((2 * O, H), jnp.float32),
    )(W1r, obj)


def _sc_gather(table, idx_main, idx_tail, Th):
    gpw = Th // NW
    gtl = gpw - GCH * C
    mesh = plsc.VectorSubcoreMesh(core_axis_name="c", subcore_axis_name="s")

    @functools.partial(
        pl.kernel,
        mesh=mesh,
        out_type=(
            jax.ShapeDtypeStruct((Th, H), jnp.float32),
            jax.ShapeDtypeStruct((Th, H), jnp.float32),
        ),
        scratch_types=[
            pltpu.VMEM((GCH, C), jnp.int32),
            pltpu.VMEM((GCH, C), jnp.int32),
            pltpu.VMEM((2, gtl), jnp.int32),
            pltpu.VMEM((C, H), jnp.float32),
            pltpu.VMEM((C, H), jnp.float32),
            pltpu.SemaphoreType.DMA,
            pltpu.SemaphoreType.DMA,
        ],
    )
    def k(table_hbm, idxm_hbm, idxt_hbm, outA_hbm, outC_hbm,
          idx_va, idx_vc, idx_vt, rows_a, rows_c, sem_a, sem_c):
        wid = lax.axis_index("s") * NC + lax.axis_index("c")
        base = wid * gpw
        pltpu.sync_copy(idxm_hbm.at[0, wid], idx_va)
        pltpu.sync_copy(idxm_hbm.at[1, wid], idx_vc)
        pltpu.sync_copy(idxt_hbm.at[0, wid], idx_vt.at[0])
        pltpu.sync_copy(idxt_hbm.at[1, wid], idx_vt.at[1])

        def body(j, carry):
            cp_a = pltpu.async_copy(table_hbm.at[idx_va.at[j]], rows_a, sem_a)
            cp_c = pltpu.async_copy(table_hbm.at[idx_vc.at[j]], rows_c, sem_c)
            cp_a.wait()
            pltpu.sync_copy(rows_a, outA_hbm.at[pl.ds(base + j * C, C)])
            cp_c.wait()
            pltpu.sync_copy(rows_c, outC_hbm.at[pl.ds(base + j * C, C)])
            return carry

        lax.fori_loop(0, GCH, body, 0)

        cp_a = pltpu.async_copy(table_hbm.at[idx_vt.at[0]],
                                rows_a.at[pl.ds(0, gtl)], sem_a)
        cp_c = pltpu.async_copy(table_hbm.at[idx_vt.at[1]],
                                rows_c.at[pl.ds(0, gtl)], sem_c)
        cp_a.wait()
        pltpu.sync_copy(rows_a.at[pl.ds(0, gtl)],
                        outA_hbm.at[pl.ds(base + GCH * C, gtl)])
        cp_c.wait()
        pltpu.sync_copy(rows_c.at[pl.ds(0, gtl)],
                        outC_hbm.at[pl.ds(base + GCH * C, gtl)])

    return k(table, idx_main, idx_tail)


def _edge_mlp(gA, gC, pred, W1b, b1r, W2, b2r, Th, joff, np_alias):
    n = Th // MT

    def body(ga, gc, pr, w1, b1_, w2, b2_, *rest):
        os_, op_, oo_ = rest[-3:]
        h = ga[...] + gc[...] + b1_[...]
        h = h + jnp.dot(pr[...].astype(jnp.bfloat16), w1[...],
                        preferred_element_type=jnp.float32)
        h = jnp.maximum(h, 0.0)
        nt = jnp.dot(h.astype(jnp.bfloat16), w2[...],
                     preferred_element_type=jnp.float32) + b2_[...]
        nt = jnp.maximum(nt, 0.0)
        os_[...] = nt[:, :H]
        op_[...] = nt[:, H:H + D]
        oo_[...] = nt[:, H + D:]

    half_spec = pl.BlockSpec((MT, D), lambda j: (j, 0))
    in_specs = [
        half_spec, half_spec,
        pl.BlockSpec((MT, D), lambda j: (joff + j, 0)),
        pl.BlockSpec((D, H), lambda j: (0, 0)),
        pl.BlockSpec((1, H), lambda j: (0, 0)),
        pl.BlockSpec((H, 2 * H + D), lambda j: (0, 0)),
        pl.BlockSpec((1, 2 * H + D), lambda j: (0, 0)),
    ]
    args = [gA, gC, pred, W1b, b1r, W2, b2r]
    io_aliases = {}
    if np_alias is not None:
        in_specs.append(pl.BlockSpec(memory_space=pl.ANY))
        args.append(np_alias)
        io_aliases = {7: 1}
    return pl.pallas_call(
        body,
        grid=(n,),
        in_specs=in_specs,
        out_specs=[
            half_spec,
            pl.BlockSpec((MT, D), lambda j: (joff + j, 0)),
            half_spec,
        ],
        out_shape=[
            jax.ShapeDtypeStruct((Th, H), jnp.float32),
            jax.ShapeDtypeStruct((T, D), jnp.float32),
            jax.ShapeDtypeStruct((Th, H), jnp.float32),
        ],
        input_output_aliases=io_aliases,
        compiler_params=pltpu.CompilerParams(
            dimension_semantics=("arbitrary",)),
    )(*args)


def _sc_scatter_add(new_s, new_o, idx_main, idx_tail, Th):
    tpc = Th // NC
    spt = tpc // NS
    sch = spt // C
    stl = spt - sch * C
    assert sch % 2 == 1
    mesh = plsc.VectorSubcoreMesh(core_axis_name="c", subcore_axis_name="s")

    @functools.partial(
        pl.kernel,
        mesh=mesh,
        out_type=jax.ShapeDtypeStruct((NC, O, H), jnp.float32),
        scratch_types=[
            pltpu.VMEM((2, sch, C), jnp.int32),
            pltpu.VMEM((2, stl), jnp.int32),
            pltpu.VMEM((2, C, PW), jnp.float32),
            pltpu.VMEM_SHARED((O, PW), jnp.float32),
            pltpu.SemaphoreType.DMA,
            pltpu.SemaphoreType.DMA,
        ],
    )
    def k(s_hbm, o_hbm, idxm_hbm, idxt_hbm, out_hbm,
          idx_v, idx_vt, rows_v, acc, semA, semB):
        c = lax.axis_index("c")
        s = lax.axis_index("s")
        base = c * tpc + s * spt

        pltpu.sync_copy(idxm_hbm.at[0, c, s], idx_v.at[0])
        pltpu.sync_copy(idxm_hbm.at[1, c, s], idx_v.at[1])
        pltpu.sync_copy(idxt_hbm.at[0, c, s], idx_vt.at[0])
        pltpu.sync_copy(idxt_hbm.at[1, c, s], idx_vt.at[1])

        for p in range(NP):
            col = pl.ds(p * PW, PW)

            def zrow(r, carry):
                for kk in range(PW // 16):
                    rows_v[0, r, pl.ds(kk * 16, 16)] = jnp.zeros(
                        (16,), jnp.float32)
                return carry

            lax.fori_loop(0, C, zrow, 0)
            z0 = 0
            for zr in (C, C, C, C, CPT - 4 * C):
                pltpu.sync_copy(rows_v.at[0, pl.ds(0, zr)],
                                acc.at[pl.ds(s * CPT + z0, zr)])
                z0 += zr

            @pl.when(s == NS - 1)
            def _():
                pltpu.sync_copy(rows_v.at[0, pl.ds(0, O - NS * CPT)],
                                acc.at[pl.ds(NS * CPT, O - NS * CPT)])

            plsc.subcore_barrier()

            for si, src_hbm in ((0, s_hbm), (1, o_hbm)):
                def cds(j):
                    return (pl.ds(base + j * C, C), col)

                pltpu.async_copy(src_hbm.at[cds(0)], rows_v.at[0], semA)

                def body2(kk, carry):
                    j0 = 2 * kk
                    pltpu.async_copy(src_hbm.at[cds(j0 + 1)],
                                     rows_v.at[1], semB)
                    pltpu.make_async_copy(src_hbm.at[cds(j0)],
                                          rows_v.at[0], semA).wait()
                    pltpu.sync_copy(rows_v.at[0],
                                    acc.at[idx_v.at[si, j0]], add=True)
                    pltpu.async_copy(src_hbm.at[cds(j0 + 2)],
                                     rows_v.at[0], semA)
                    pltpu.make_async_copy(src_hbm.at[cds(j0 + 1)],
                                          rows_v.at[1], semB).wait()
                    pltpu.sync_copy(rows_v.at[1],
                                    acc.at[idx_v.at[si, j0 + 1]], add=True)
                    return carry

                lax.fori_loop(0, sch // 2, body2, 0)
                pltpu.make_async_copy(src_hbm.at[cds(sch - 1)],
                                      rows_v.at[0], semA).wait()
                pltpu.sync_copy(rows_v.at[0],
                                acc.at[idx_v.at[si, sch - 1]], add=True)
                pltpu.sync_copy(
                    src_hbm.at[pl.ds(base + sch * C, stl), col],
                    rows_v.at[0, pl.ds(0, stl)])
                pltpu.sync_copy(rows_v.at[0, pl.ds(0, stl)],
                                acc.at[idx_vt.at[si]], add=True)

            plsc.subcore_barrier()

            pltpu.sync_copy(acc.at[pl.ds(s * CPT, CPT)],
                            out_hbm.at[c, pl.ds(s * CPT, CPT), col])

            @pl.when(s == NS - 1)
            def _():
                pltpu.sync_copy(
                    acc.at[pl.ds(NS * CPT, O - NS * CPT)],
                    out_hbm.at[c, pl.ds(NS * CPT, O - NS * CPT), col])

            plsc.subcore_barrier()

    return k(new_s, new_o, idx_main, idx_tail)


def _sumsq(pa, pb):
    n = 25
    b = O // n

    def body(a_ref, b_ref, o_ref, acc_ref):
        @pl.when(pl.program_id(0) == 0)
        def _():
            acc_ref[0] = 0.0

        x = a_ref[0] + a_ref[1] + b_ref[0] + b_ref[1]
        acc_ref[0] += jnp.sum(x * x)

        @pl.when(pl.program_id(0) == n - 1)
        def _():
            o_ref[...] = jnp.broadcast_to(acc_ref[0], (1, 1))

    p_spec = pl.BlockSpec((NC, b, H), lambda j: (0, j, 0))
    return pl.pallas_call(
        body,
        grid=(n,),
        in_specs=[p_spec, p_spec],
        out_specs=pl.BlockSpec((1, 1), lambda j: (0, 0)),
        out_shape=jax.ShapeDtypeStruct((1, 1), jnp.float32),
        scratch_shapes=[pltpu.SMEM((1,), jnp.float32)],
        compiler_params=pltpu.CompilerParams(
            dimension_semantics=("arbitrary",)),
    )(pa, pb)


def _gconv2(pa, pb, ss, W3, b3r, W4, b4r):
    n = 25
    b = O // n

    def body(ss_ref, a_ref, b_ref, w3, b3_, w4, b4_, o_ref):
        inv = lax.rsqrt(ss_ref[0, 0])
        x = (a_ref[0] + a_ref[1] + b_ref[0] + b_ref[1]) * inv
        h = jnp.dot(x, w3[...],
                    preferred_element_type=jnp.float32) + b3_[...]
        h = jnp.maximum(h, 0.0)
        o = jnp.dot(h, w4[...], preferred_element_type=jnp.float32) + b4_[...]
        o_ref[...] = jnp.maximum(o, 0.0)

    p_spec = pl.BlockSpec((NC, b, H), lambda j: (0, j, 0))
    return pl.pallas_call(
        body,
        grid=(n,),
        in_specs=[
            pl.BlockSpec((1, 1), lambda j: (0, 0)),
            p_spec, p_spec,
            pl.BlockSpec((H, H), lambda j: (0, 0)),
            pl.BlockSpec((1, H), lambda j: (0, 0)),
            pl.BlockSpec((H, D), lambda j: (0, 0)),
            pl.BlockSpec((1, D), lambda j: (0, 0)),
        ],
        out_specs=pl.BlockSpec((b, D), lambda j: (j, 0)),
        out_shape=jax.ShapeDtypeStruct((O, D), jnp.float32),
    )(ss, pa, pb, W3, b3r, W4, b4r)


def _gather_idx(s_h, o_h, Th):
    gpw = Th // NW
    idx = jnp.stack([s_h, o_h + O]).reshape(2, NW, gpw)
    return (idx[:, :, :GCH * C].reshape(2, NW, GCH, C), idx[:, :, GCH * C:])


def _scatter_idx(s_h, o_h, Th):
    spt = Th // NC // NS
    sch = spt // C
    idx = jnp.stack([s_h, o_h]).reshape(2, NC, NS, spt)
    return (idx[..., :sch * C].reshape(2, NC, NS, sch, C), idx[..., sch * C:])


def kernel(obj_vecs, pred_vecs, edges, W1, b1, W2, b2, W3, b3, W4, b4):
    obj = obj_vecs[0]
    pred = pred_vecs[0]
    s_idx = edges[0, :, 0].astype(jnp.int32)
    o_idx = edges[0, :, 1].astype(jnp.int32)

    W1r = W1.reshape(3, D, H)
    W1b = W1r[1].astype(jnp.bfloat16)
    W2b = W2.astype(jnp.bfloat16)
    b1r = b1.reshape(1, H)
    b2r = b2.reshape(1, 2 * H + D)

    table = _proj_table(obj, W1r)

    parts = []
    np_alias = None
    for t0, Th in ((0, TA), (TA, TB)):
        s_h = lax.dynamic_slice_in_dim(s_idx, t0, Th)
        o_h = lax.dynamic_slice_in_dim(o_idx, t0, Th)
        gm, gt = _gather_idx(s_h, o_h, Th)
        gA, gC = _sc_gather(table, gm, gt, Th)
        new_s, np_alias, new_o = _edge_mlp(
            gA, gC, pred, W1b, b1r, W2b, b2r, Th, t0 // MT, np_alias)
        sm, st = _scatter_idx(s_h, o_h, Th)
        parts.append(_sc_scatter_add(new_s, new_o, sm, st, Th))
    new_pred = np_alias

    ss = _sumsq(parts[0], parts[1])
    new_obj = _gconv2(parts[0], parts[1], ss,
                      W3, b3.reshape(1, H), W4, b4.reshape(1, D))

    return new_obj[None], new_pred[None]


# speedup vs baseline: 1.1006x; 1.1006x over previous
import functools

import jax
import jax.numpy as jnp
from jax import lax
from jax.experimental import pallas as pl
from jax.experimental.pallas import tpu as pltpu
from jax.experimental.pallas import tpu_sc as plsc

O = 10000
T = 160000
D = 384
H = 384

NC = 2
NS = 16
NW = NC * NS

C = 128
GC = 64
GPW = T // NW
GCH = GPW // GC
GTL = GPW - GCH * GC
TPC = T // NC
SPT = TPC // NS
SCH = SPT // C
STL = SPT - SCH * C
PW = 128
NP = H // PW
CPT = 624


def _proj_table(obj, W1r):
    nO = 10
    bO = O // nO

    def body(w_ref, x_ref, o_ref):
        o_ref[...] = jnp.dot(x_ref[...], w_ref[0],
                             preferred_element_type=jnp.float32)

    return pl.pallas_call(
        body,
        grid=(2, nO),
        in_specs=[
            pl.BlockSpec((1, D, H), lambda g, j: (2 * g, 0, 0)),
            pl.BlockSpec((bO, D), lambda g, j: (j, 0)),
        ],
        out_specs=pl.BlockSpec((bO, H), lambda g, j: (g * nO + j, 0)),
        out_shape=jax.ShapeDtypeStruct((2 * O, H), jnp.float32),
    )(W1r, obj)


def _sc_gather(table, idx_main, idx_tail):
    mesh = plsc.VectorSubcoreMesh(core_axis_name="c", subcore_axis_name="s")

    @functools.partial(
        pl.kernel,
        mesh=mesh,
        out_type=(
            jax.ShapeDtypeStruct((T, H), jnp.float32),
            jax.ShapeDtypeStruct((T, H), jnp.float32),
        ),
        scratch_types=[
            pltpu.VMEM((GCH, GC), jnp.int32),
            pltpu.VMEM((GCH, GC), jnp.int32),
            pltpu.VMEM((2, GTL), jnp.int32),
            pltpu.VMEM((2, GC, H), jnp.float32),
            pltpu.VMEM((2, GC, H), jnp.float32),
            [pltpu.SemaphoreType.DMA] * 4,
            [pltpu.SemaphoreType.DMA] * 4,
        ],
    )
    def k(table_hbm, idxm_hbm, idxt_hbm, outA_hbm, outC_hbm,
          idx_va, idx_vc, idx_vt, rows_a, rows_c, sem_r, sem_w):
        wid = lax.axis_index("s") * NC + lax.axis_index("c")
        base = wid * GPW
        pltpu.sync_copy(idxm_hbm.at[0, wid], idx_va)
        pltpu.sync_copy(idxm_hbm.at[1, wid], idx_vc)
        pltpu.sync_copy(idxt_hbm.at[0, wid], idx_vt.at[0])
        pltpu.sync_copy(idxt_hbm.at[1, wid], idx_vt.at[1])

        slots = ((idx_va, rows_a, outA_hbm, 0), (idx_vc, rows_c, outC_hbm, 1))

        for b in range(2):
            for idx_v, rows, _, t in slots:
                pltpu.async_copy(table_hbm.at[idx_v.at[b]], rows.at[b],
                                 sem_r[2 * b + t])

        def body(kk, carry):
            for b in range(2):
                j = 2 * kk + b
                for idx_v, rows, out_hbm, t in slots:
                    pltpu.make_async_copy(table_hbm.at[idx_v.at[j]],
                                          rows.at[b], sem_r[2 * b + t]).wait()
                    pltpu.async_copy(rows.at[b],
                                     out_hbm.at[pl.ds(base + j * GC, GC)],
                                     sem_w[2 * b + t])
            for b in range(2):
                j2 = 2 * kk + b + 2
                for idx_v, rows, out_hbm, t in slots:
                    pltpu.make_async_copy(
                        rows.at[b],
                        out_hbm.at[pl.ds(base + (j2 - 2) * GC, GC)],
                        sem_w[2 * b + t]).wait()

                    @pl.when(j2 < GCH)
                    def _():
                        pltpu.async_copy(table_hbm.at[idx_v.at[j2]],
                                         rows.at[b], sem_r[2 * b + t])
            return carry

        lax.fori_loop(0, GCH // 2, body, 0)

        cp_a = pltpu.async_copy(table_hbm.at[idx_vt.at[0]],
                                rows_a.at[0, pl.ds(0, GTL)], sem_r[0])
        cp_c = pltpu.async_copy(table_hbm.at[idx_vt.at[1]],
                                rows_c.at[0, pl.ds(0, GTL)], sem_r[1])
        cp_a.wait()
        pltpu.sync_copy(rows_a.at[0, pl.ds(0, GTL)],
                        outA_hbm.at[pl.ds(base + GCH * GC, GTL)])
        cp_c.wait()
        pltpu.sync_copy(rows_c.at[0, pl.ds(0, GTL)],
                        outC_hbm.at[pl.ds(base + GCH * GC, GTL)])

    return k(table, idx_main, idx_tail)


def _edge_mlp(gA, gC, pred, W1b, b1r, W2, b2r):
    tile = 640
    n = T // tile

    def body(ga, gc, pr, w1, b1_, w2, b2_, os_, op_, oo_):
        h = ga[...] + gc[...] + b1_[...]
        h = h + jnp.dot(pr[...].astype(jnp.bfloat16), w1[...],
                        preferred_element_type=jnp.float32)
        h = jnp.maximum(h, 0.0)
        nt = jnp.dot(h.astype(jnp.bfloat16), w2[...],
                     preferred_element_type=jnp.float32) + b2_[...]
        nt = jnp.maximum(nt, 0.0)
        os_[...] = nt[:, :H]
        op_[...] = nt[:, H:H + D]
        oo_[...] = nt[:, H + D:]

    edge_spec = pl.BlockSpec((tile, D), lambda j: (j, 0))
    return pl.pallas_call(
        body,
        grid=(n,),
        in_specs=[
            edge_spec, edge_spec, edge_spec,
            pl.BlockSpec((D, H), lambda j: (0, 0)),
            pl.BlockSpec((1, H), lambda j: (0, 0)),
            pl.BlockSpec((H, 2 * H + D), lambda j: (0, 0)),
            pl.BlockSpec((1, 2 * H + D), lambda j: (0, 0)),
        ],
        out_specs=[edge_spec, edge_spec, edge_spec],
        out_shape=[
            jax.ShapeDtypeStruct((T, H), jnp.float32),
            jax.ShapeDtypeStruct((T, D), jnp.float32),
            jax.ShapeDtypeStruct((T, H), jnp.float32),
        ],
        compiler_params=pltpu.CompilerParams(
            dimension_semantics=("arbitrary",)),
    )(gA, gC, pred, W1b, b1r, W2, b2r)


def _sc_scatter_add(new_s, new_o, idx_main, idx_tail):
    mesh = plsc.VectorSubcoreMesh(core_axis_name="c", subcore_axis_name="s")

    @functools.partial(
        pl.kernel,
        mesh=mesh,
        out_type=jax.ShapeDtypeStruct((NC, O, H), jnp.float32),
        scratch_types=[
            pltpu.VMEM((2, SCH, C), jnp.int32),
            pltpu.VMEM((2, STL), jnp.int32),
            pltpu.VMEM((2, C, PW), jnp.float32),
            pltpu.VMEM_SHARED((O, PW), jnp.float32),
            pltpu.SemaphoreType.DMA,
            pltpu.SemaphoreType.DMA,
        ],
    )
    def k(s_hbm, o_hbm, idxm_hbm, idxt_hbm, out_hbm,
          idx_v, idx_vt, rows_v, acc, semA, semB):
        c = lax.axis_index("c")
        s = lax.axis_index("s")
        base = c * TPC + s * SPT

        pltpu.sync_copy(idxm_hbm.at[0, c, s], idx_v.at[0])
        pltpu.sync_copy(idxm_hbm.at[1, c, s], idx_v.at[1])
        pltpu.sync_copy(idxt_hbm.at[0, c, s], idx_vt.at[0])
        pltpu.sync_copy(idxt_hbm.at[1, c, s], idx_vt.at[1])

        for p in range(NP):
            col = pl.ds(p * PW, PW)

            def zrow(r, carry):
                for kk in range(PW // 16):
                    rows_v[0, r, pl.ds(kk * 16, 16)] = jnp.zeros(
                        (16,), jnp.float32)
                return carry

            lax.fori_loop(0, C, zrow, 0)
            z0 = 0
            for zr in (C, C, C, C, CPT - 4 * C):
                pltpu.sync_copy(rows_v.at[0, pl.ds(0, zr)],
                                acc.at[pl.ds(s * CPT + z0, zr)])
                z0 += zr

            @pl.when(s == NS - 1)
            def _():
                pltpu.sync_copy(rows_v.at[0, pl.ds(0, O - NS * CPT)],
                                acc.at[pl.ds(NS * CPT, O - NS * CPT)])

            plsc.subcore_barrier()

            for si, src_hbm in ((0, s_hbm), (1, o_hbm)):
                def cds(j):
                    return (pl.ds(base + j * C, C), col)

                pltpu.async_copy(src_hbm.at[cds(0)], rows_v.at[0], semA)

                def body2(kk, carry):
                    j0 = 2 * kk
                    pltpu.async_copy(src_hbm.at[cds(j0 + 1)],
                                     rows_v.at[1], semB)
                    pltpu.make_async_copy(src_hbm.at[cds(j0)],
                                          rows_v.at[0], semA).wait()
                    pltpu.sync_copy(rows_v.at[0],
                                    acc.at[idx_v.at[si, j0]], add=True)
                    pltpu.async_copy(src_hbm.at[cds(j0 + 2)],
                                     rows_v.at[0], semA)
                    pltpu.make_async_copy(src_hbm.at[cds(j0 + 1)],
                                          rows_v.at[1], semB).wait()
                    pltpu.sync_copy(rows_v.at[1],
                                    acc.at[idx_v.at[si, j0 + 1]], add=True)
                    return carry

                lax.fori_loop(0, SCH // 2, body2, 0)
                pltpu.make_async_copy(src_hbm.at[cds(SCH - 1)],
                                      rows_v.at[0], semA).wait()
                pltpu.sync_copy(rows_v.at[0],
                                acc.at[idx_v.at[si, SCH - 1]], add=True)
                pltpu.sync_copy(
                    src_hbm.at[pl.ds(base + SCH * C, STL), col],
                    rows_v.at[0, pl.ds(0, STL)])
                pltpu.sync_copy(rows_v.at[0, pl.ds(0, STL)],
                                acc.at[idx_vt.at[si]], add=True)

            plsc.subcore_barrier()

            pltpu.sync_copy(acc.at[pl.ds(s * CPT, CPT)],
                            out_hbm.at[c, pl.ds(s * CPT, CPT), col])

            @pl.when(s == NS - 1)
            def _():
                pltpu.sync_copy(
                    acc.at[pl.ds(NS * CPT, O - NS * CPT)],
                    out_hbm.at[c, pl.ds(NS * CPT, O - NS * CPT), col])

            plsc.subcore_barrier()

    return k(new_s, new_o, idx_main, idx_tail)


def _gconv2(partial, W3, b3r, W4, b4r):
    n = 25
    b = O // n

    def body(x_ref, w3, b3_, w4, b4_, o_ref, acc_ref):
        ph = pl.program_id(0)

        @pl.when((ph == 0) & (pl.program_id(1) == 0))
        def _():
            acc_ref[0] = 0.0

        @pl.when(ph == 0)
        def _():
            x = x_ref[0] + x_ref[1]
            acc_ref[0] += jnp.sum(x * x)

        @pl.when(ph == 1)
        def _():
            inv = lax.rsqrt(acc_ref[0])
            x = (x_ref[0] + x_ref[1]) * inv
            h = jnp.dot(x, w3[...],
                        preferred_element_type=jnp.float32) + b3_[...]
            h = jnp.maximum(h, 0.0)
            o = jnp.dot(h, w4[...],
                        preferred_element_type=jnp.float32) + b4_[...]
            o_ref[...] = jnp.maximum(o, 0.0)

    return pl.pallas_call(
        body,
        grid=(2, n),
        in_specs=[
            pl.BlockSpec((NC, b, H), lambda p, j: (0, j, 0)),
            pl.BlockSpec((H, H), lambda p, j: (0, 0)),
            pl.BlockSpec((1, H), lambda p, j: (0, 0)),
            pl.BlockSpec((H, D), lambda p, j: (0, 0)),
            pl.BlockSpec((1, D), lambda p, j: (0, 0)),
        ],
        out_specs=pl.BlockSpec((b, D), lambda p, j: (j, 0)),
        out_shape=jax.ShapeDtypeStruct((O, D), jnp.float32),
        scratch_shapes=[pltpu.SMEM((1,), jnp.float32)],
        compiler_params=pltpu.CompilerParams(
            dimension_semantics=("arbitrary", "arbitrary")),
    )(partial, W3, b3r, W4, b4r)


def kernel(obj_vecs, pred_vecs, edges, W1, b1, W2, b2, W3, b3, W4, b4):
    obj = obj_vecs[0]
    pred = pred_vecs[0]
    s_idx = edges[0, :, 0].astype(jnp.int32)
    o_idx = edges[0, :, 1].astype(jnp.int32)

    W1r = W1.reshape(3, D, H)

    table = _proj_table(obj, W1r)

    idx_g = jnp.stack([s_idx, o_idx + O]).reshape(2, NW, GPW)
    idx_gm = idx_g[:, :, :GCH * GC].reshape(2, NW, GCH, GC)
    idx_gt = idx_g[:, :, GCH * GC:]
    gA, gC = _sc_gather(table, idx_gm, idx_gt)

    new_s, new_pred, new_o = _edge_mlp(
        gA, gC, pred, W1r[1].astype(jnp.bfloat16), b1.reshape(1, H),
        W2.astype(jnp.bfloat16), b2.reshape(1, 2 * H + D))

    idx_sc = jnp.stack([s_idx, o_idx]).reshape(2, NC, NS, SPT)
    idx_sm = idx_sc[..., :SCH * C].reshape(2, NC, NS, SCH, C)
    idx_st = idx_sc[..., SCH * C:]
    partial = _sc_scatter_add(new_s, new_o, idx_sm, idx_st)

    new_obj = _gconv2(partial, W3, b3.reshape(1, H), W4, b4.reshape(1, D))

    return new_obj[None], new_pred[None]
